# Initial kernel scaffold; baseline (speedup 1.0000x reference)
import jax
import jax.numpy as jnp
from jax.experimental import pallas as pl


def kernel(x, edge_index, edge_attr, batch, atom_emb, bond_emb, W, b, ln_g, ln_b):
    raise NotImplementedError("write your pallas kernel here")


# speedup vs baseline: 2.3908x; 2.3908x over previous
import functools

import jax
import jax.numpy as jnp
from jax import lax
from jax.experimental import pallas as pl
from jax.experimental.pallas import tpu as pltpu
from jax.experimental.pallas import tpu_sc as plsc

_N = 10000
_NP = 10240
_E = 320000
_EP = 327680
_H = 128
_LAYERS = 7
_EPS = 1e-7
_CH = 128
_ECHUNKS = _EP // _CH


@functools.lru_cache(maxsize=None)
def _sc_info():
    info = plsc.get_sparse_core_info()
    return info.num_cores, info.num_subcores


@functools.lru_cache(maxsize=None)
def _build_sc_encode():
    nc, ns = _sc_info()
    ntiles = nc * ns
    nchunk = _ECHUNKS // ntiles
    rows_per_tile = _NP // ntiles
    xrows = _NP // 40
    crows = 512 // ntiles
    mesh = plsc.VectorSubcoreMesh(core_axis_name="c", subcore_axis_name="s")

    def body(x9r, a0, a1, a2, a3, a4, a5, a6, a7, a8, bcat, ea0, ea1, ea2,
             henc, ctab, eidx,
             idxf, abuf, hacc, bb, ctb, e0v, e1v, e2v, sem):
        aembs = [a0, a1, a2, a3, a4, a5, a6, a7, a8]
        cid = lax.axis_index("c")
        sid = lax.axis_index("s")
        wid = cid * ns + sid

        pltpu.sync_copy(bcat, bb)

        @pl.loop(0, crows)
        def _(t):
            i = wid * crows + t
            r0 = i & 7
            r1 = 8 + ((i >> 3) & 7)
            r2 = 16 + ((i >> 6) & 7)
            for c in range(_H // 16):
                sl = pl.ds(c * 16, 16)
                ctb[t, sl] = bb[r0, sl] + bb[r1, sl] + bb[r2, sl]

        pltpu.sync_copy(ctb, ctab.at[pl.ds(wid * crows, crows)])

        ebase = wid * nchunk
        pltpu.sync_copy(ea0.at[pl.ds(ebase, nchunk)], e0v)
        pltpu.sync_copy(ea1.at[pl.ds(ebase, nchunk)], e1v)
        pltpu.sync_copy(ea2.at[pl.ds(ebase, nchunk)], e2v)

        @pl.loop(0, nchunk)
        def _(r):
            for c in range(_CH // 16):
                sl = pl.ds(c * 16, 16)
                e0v[r, sl] = e0v[r, sl] + (e1v[r, sl] << 3) + (e2v[r, sl] << 6)

        pltpu.sync_copy(e0v, eidx.at[pl.ds(ebase, nchunk)])

        for f in range(9):
            pltpu.sync_copy(x9r.at[pl.ds(f * xrows + wid * 8, 8)], idxf)
            for k in range(8):
                if f == 0:
                    pltpu.async_copy(
                        aembs[f].at[idxf.at[k]],
                        hacc.at[pl.ds(k * 40, 40)], sem).wait()
                else:
                    pltpu.async_copy(aembs[f].at[idxf.at[k]], abuf, sem).wait()

                    @pl.loop(0, 40)
                    def _(r):
                        for c in range(_H // 16):
                            sl = pl.ds(c * 16, 16)
                            hacc[k * 40 + r, sl] = hacc[k * 40 + r, sl] + abuf[r, sl]

        pltpu.sync_copy(hacc, henc.at[pl.ds(wid * rows_per_tile, rows_per_tile)])

    return pl.kernel(
        body,
        out_type=(
            jax.ShapeDtypeStruct((_NP, _H), jnp.float32),
            jax.ShapeDtypeStruct((512, _H), jnp.float32),
            jax.ShapeDtypeStruct((_ECHUNKS, _CH), jnp.int32),
        ),
        mesh=mesh,
        scratch_types=(
            pltpu.VMEM((8, 40), jnp.int32),
            pltpu.VMEM((40, _H), jnp.float32),
            pltpu.VMEM((rows_per_tile, _H), jnp.float32),
            pltpu.VMEM((24, _H), jnp.float32),
            pltpu.VMEM((crows, _H), jnp.float32),
            pltpu.VMEM((nchunk, _CH), jnp.int32),
            pltpu.VMEM((nchunk, _CH), jnp.int32),
            pltpu.VMEM((nchunk, _CH), jnp.int32),
            pltpu.SemaphoreType.DMA,
        ),
    )


@functools.lru_cache(maxsize=None)
def _build_sc_msg():
    nc, ns = _sc_info()
    ntiles = nc * ns
    nchunk = _ECHUNKS // ntiles
    mrows = _NP // ns
    mesh = plsc.VectorSubcoreMesh(core_axis_name="c", subcore_axis_name="s")

    def body(hcur, srcr, eidxr, dstr, ctab,
             mout,
             srcv, eiv, dstv, hbuf, eebuf, msh, sem1, sem2):
        cid = lax.axis_index("c")
        sid = lax.axis_index("s")
        wid = cid * ns + sid
        base = wid * nchunk

        @pl.loop(0, _CH)
        def _(r):
            for c in range(_H // 16):
                hbuf[r, pl.ds(c * 16, 16)] = jnp.zeros((16,), jnp.float32)

        mbase = sid * mrows
        for k in range(mrows // _CH):
            pltpu.sync_copy(hbuf, msh.at[pl.ds(mbase + k * _CH, _CH)])
        plsc.subcore_barrier()

        @pl.loop(0, nchunk // 8)
        def _(g):
            gb = base + g * 8
            pltpu.sync_copy(srcr.at[pl.ds(gb, 8)], srcv)
            pltpu.sync_copy(eidxr.at[pl.ds(gb, 8)], eiv)
            pltpu.sync_copy(dstr.at[pl.ds(gb, 8)], dstv)

            @pl.loop(0, 8)
            def _(j):
                cp1 = pltpu.async_copy(hcur.at[srcv.at[j]], hbuf, sem1)
                cp2 = pltpu.async_copy(ctab.at[eiv.at[j]], eebuf, sem2)
                cp1.wait()
                cp2.wait()

                @pl.loop(0, _CH)
                def _(r):
                    for c in range(_H // 16):
                        sl = pl.ds(c * 16, 16)
                        v = hbuf[r, sl] + eebuf[r, sl]
                        hbuf[r, sl] = jnp.maximum(v, 0.0) + _EPS

                pltpu.sync_copy(hbuf, msh.at[dstv.at[j]], add=True)

        plsc.subcore_barrier()
        pltpu.sync_copy(msh.at[pl.ds(mbase, mrows)],
                        mout.at[cid, pl.ds(mbase, mrows)])

    return pl.kernel(
        body,
        out_type=jax.ShapeDtypeStruct((nc, _NP, _H), jnp.float32),
        mesh=mesh,
        scratch_types=(
            pltpu.VMEM((8, _CH), jnp.int32),
            pltpu.VMEM((8, _CH), jnp.int32),
            pltpu.VMEM((8, _CH), jnp.int32),
            pltpu.VMEM((_CH, _H), jnp.float32),
            pltpu.VMEM((_CH, _H), jnp.float32),
            pltpu.VMEM_SHARED((_NP, _H), jnp.float32),
            pltpu.SemaphoreType.DMA,
            pltpu.SemaphoreType.DMA,
        ),
    )


@functools.lru_cache(maxsize=None)
def _build_tc_layer(nc, with_res, final):
    blk = 256
    grid = _NP // blk

    def body(*refs):
        if with_res:
            h2, m, res, w, bv, g, bt, out_h, out_aux = refs
        else:
            h2, m, w, bv, g, bt, out_h, out_aux = refs
        t = h2[...]
        for c in range(nc):
            t = t + m[c]
        y = jnp.dot(t, w[...], preferred_element_type=jnp.float32) + bv[...]
        if with_res:
            y = y + res[...]
        out_h[...] = y
        mu = jnp.mean(y, axis=-1, keepdims=True)
        var = jnp.mean((y - mu) ** 2, axis=-1, keepdims=True)
        z = (y - mu) * lax.rsqrt(var + 1e-5) * g[...] + bt[...]
        if not final:
            z = jnp.maximum(z, 0.0)
        out_aux[...] = z

    row_spec = pl.BlockSpec((blk, _H), lambda i: (i, 0))
    m_spec = pl.BlockSpec((nc, blk, _H), lambda i: (0, i, 0))
    full_spec = pl.BlockSpec((_H, _H), lambda i: (0, 0))
    vec_spec = pl.BlockSpec((1, _H), lambda i: (0, 0))
    in_specs = [row_spec, m_spec]
    if with_res:
        in_specs.append(row_spec)
    in_specs += [full_spec, vec_spec, vec_spec, vec_spec]
    return pl.pallas_call(
        body,
        grid=(grid,),
        in_specs=in_specs,
        out_specs=[row_spec, row_spec],
        out_shape=[
            jax.ShapeDtypeStruct((_NP, _H), jnp.float32),
            jax.ShapeDtypeStruct((_NP, _H), jnp.float32),
        ],
    )


def kernel(x, edge_index, edge_attr, batch, atom_emb, bond_emb, W, b, ln_g, ln_b):
    nc, _ = _sc_info()
    x9r = jnp.pad(x, ((0, _NP - _N), (0, 0))).T.reshape(9 * (_NP // 40), 40)
    aembs = [atom_emb[f] for f in range(9)]
    bcat = bond_emb.reshape(24, _H)
    epad = _EP - _E
    src = jnp.pad(edge_index[0], (0, epad)).reshape(_ECHUNKS, _CH)
    dst = jnp.pad(edge_index[1], (0, epad),
                  constant_values=_N).reshape(_ECHUNKS, _CH)
    ea0 = jnp.pad(edge_attr[:, 0], (0, epad)).reshape(_ECHUNKS, _CH)
    ea1 = jnp.pad(edge_attr[:, 1], (0, epad)).reshape(_ECHUNKS, _CH)
    ea2 = jnp.pad(edge_attr[:, 2], (0, epad)).reshape(_ECHUNKS, _CH)

    sc_encode = _build_sc_encode()
    sc_msg = _build_sc_msg()

    henc, ctab, eidx = sc_encode(x9r, *aembs, bcat, ea0, ea1, ea2)

    m = sc_msg(henc, src, eidx, dst, ctab)
    h, aux = _build_tc_layer(nc, False, False)(
        henc, m, W[0], b[0:1], ln_g[0:1], ln_b[0:1])

    for l in range(1, _LAYERS):
        m = sc_msg(aux, src, eidx, dst, ctab)
        final = l == _LAYERS - 1
        h, aux = _build_tc_layer(nc, True, final)(
            aux, m, h, W[l], b[l:l + 1], ln_g[l:l + 1], ln_b[l:l + 1])

    return aux[:_N]
